# baseline (speedup 1.0000x reference)
import functools

import jax
import jax.numpy as jnp
from jax import lax
from jax.experimental import pallas as pl
from jax.experimental.pallas import tpu as pltpu
from jax.experimental.pallas import tpu_sc as plsc

_NC = 2
_NS = 16
_NW = _NC * _NS

_CROWS = 32
_NBUF = 2
_SLICES = [0, 16, 32, 48, 64, 80, 96, 112, 128, 144, 160, 176, 184]


def _make_sc_gather(n_tab, n_rows, n_cols):
    rows_per_tile = n_rows // _NW
    n_chunks = rows_per_tile // _CROWS
    chunk = _CROWS * n_cols
    mesh = plsc.VectorSubcoreMesh(core_axis_name="c", subcore_axis_name="s")

    @functools.partial(
        pl.kernel,
        mesh=mesh,
        out_type=jax.ShapeDtypeStruct((n_rows * n_cols,), jnp.float32),
        scratch_types=[pltpu.VMEM_SHARED((n_tab,), jnp.float32)]
        + [pltpu.VMEM((_CROWS, n_cols), jnp.int32)] * _NBUF
        + [pltpu.VMEM((chunk,), jnp.int32)] * _NBUF
        + [pltpu.VMEM((chunk,), jnp.float32)] * _NBUF
        + [pltpu.VMEM((_CROWS, n_cols), jnp.float32)] * _NBUF
        + [pltpu.SemaphoreType.DMA] * (3 * _NBUF),
    )
    def gather_kernel(table_hbm, idx_hbm, out_hbm, spm, *rest):
        idx2d = rest[0:_NBUF]
        idx1d = rest[_NBUF : 2 * _NBUF]
        rows1d = rest[2 * _NBUF : 3 * _NBUF]
        rows2d = rest[3 * _NBUF : 4 * _NBUF]
        sems = rest[4 * _NBUF :]
        sem_i = sems[0:_NBUF]
        sem_g = sems[_NBUF : 2 * _NBUF]
        sem_o = sems[2 * _NBUF :]
        s = lax.axis_index("s")
        wid = s * _NC + lax.axis_index("c")

        @pl.when(s == 0)
        def _stage():
            pltpu.sync_copy(table_hbm, spm)

        plsc.subcore_barrier()
        rbase = wid * rows_per_tile

        def idx_start(i):
            b = i % _NBUF
            src = idx_hbm.at[pl.ds(rbase + i * _CROWS, _CROWS), :]
            return pltpu.async_copy(src, idx2d[b], sem_i[b])

        def flatten_idx(b):
            def row(r, _):
                for c0 in _SLICES:
                    idx1d[b][pl.ds(r * n_cols + c0, 16)] = idx2d[b][r, pl.ds(c0, 16)]
                return ()

            lax.fori_loop(0, _CROWS, row, ())

        def gather_start(i):
            b = i % _NBUF
            return pltpu.async_copy(spm.at[idx1d[b]], rows1d[b], sem_g[b])

        def elu_unflatten(b):
            def body(j, _):
                v = rows1d[b][pl.ds(j * 16, 16)]
                rows1d[b][pl.ds(j * 16, 16)] = jnp.where(v > 0, v + 1.0, jnp.exp(v))
                return ()

            lax.fori_loop(0, chunk // 16, body, ())

        def out_start(i):
            b = i % _NBUF
            dst = out_hbm.at[pl.ds((rbase + i * _CROWS) * n_cols, chunk)]
            return pltpu.async_copy(rows1d[b], dst, sem_o[b])

        cp = {0: idx_start(0)}
        g = {}
        o = {}
        for i in range(n_chunks):
            b = i % _NBUF
            cp[i].wait()
            flatten_idx(b)
            if i >= _NBUF:
                o[i - _NBUF].wait()
            g[i] = gather_start(i)
            if i >= 1:
                g[i - 1].wait()
                if i + 1 < n_chunks:
                    cp[i + 1] = idx_start(i + 1)
                elu_unflatten((i - 1) % _NBUF)
                o[i - 1] = out_start(i - 1)
            elif i + 1 < n_chunks:
                cp[i + 1] = idx_start(i + 1)
        g[n_chunks - 1].wait()
        elu_unflatten((n_chunks - 1) % _NBUF)
        o[n_chunks - 1] = out_start(n_chunks - 1)
        for j in range(max(0, n_chunks - _NBUF), n_chunks):
            o[j].wait()

    return gather_kernel


def kernel(points, uncertainty):
    b, t = points.shape
    table = uncertainty.reshape(-1)
    out = _make_sc_gather(table.shape[0], b, t)(table, points)
    return out.reshape(b, t, 1)
